# baseline (speedup 1.0000x reference)
import dataclasses
import functools

import jax
import jax.numpy as jnp
from jax import lax
from jax.experimental import pallas as pl
from jax.experimental.pallas import tpu as pltpu
from jax.experimental.pallas import tpu_sc as plsc

N = 10000
E = 160000
IN = 128
HID = 8
OUT = 128
D_E = 16

NC = 2
NS = 16
NW = NC * NS
EPW = E // NW
CH = 40
NCHUNK = EPW // CH

NPAD = 10240
ROWS_PER_TILE = NPAD // NS
MSG_W = 16

EB = 1280
NB = 2000

CHG = 40
NFULL = EPW // CHG
ROW_W = IN


def _gather_rows(x_i32, src):
    mesh = plsc.VectorSubcoreMesh(core_axis_name="c", subcore_axis_name="s")

    @functools.partial(
        pl.kernel,
        mesh=mesh,
        out_type=jax.ShapeDtypeStruct((E, ROW_W), jnp.int32),
        scratch_types=[
            pltpu.VMEM((EPW,), jnp.int32),
            pltpu.VMEM((CHG, ROW_W), jnp.int32),
        ],
    )
    def body(x_hbm, src_hbm, xj_hbm, idx_v, buf_v):
        cid = lax.axis_index("c")
        sid = lax.axis_index("s")
        wid = sid * NC + cid
        base = wid * EPW

        pltpu.sync_copy(src_hbm.at[pl.ds(base, EPW)], idx_v)

        @pl.loop(0, NFULL)
        def _(k):
            pltpu.sync_copy(x_hbm.at[idx_v.at[pl.ds(k * CHG, CHG)]], buf_v)
            pltpu.sync_copy(buf_v, xj_hbm.at[pl.ds(base + k * CHG, CHG)])

    return body(x_i32, src)


def _scatter_add(msg_flat, dst):
    mesh = plsc.VectorSubcoreMesh(core_axis_name="c", subcore_axis_name="s")
    LANES = 16
    EPAIR = EPW // 2
    cp = pltpu.CompilerParams()
    if "needs_layout_passes" in pltpu.CompilerParams.__dataclass_fields__:
        cp = dataclasses.replace(cp, needs_layout_passes=False)

    @functools.partial(
        pl.kernel,
        mesh=mesh,
        compiler_params=cp,
        out_type=jax.ShapeDtypeStruct((NW, N * HID), jnp.float32),
        scratch_types=[
            pltpu.VMEM((EPW,), jnp.int32),
            pltpu.VMEM((EPW * HID,), jnp.float32),
            pltpu.VMEM((N * HID,), jnp.float32),
        ],
    )
    def body(msg_hbm, dst_hbm, agg_hbm, dst_v, msg_v, acc_v):
        cid = lax.axis_index("c")
        sid = lax.axis_index("s")
        wid = sid * NC + cid

        pltpu.sync_copy(dst_hbm.at[pl.ds(wid * EPW, EPW)], dst_v)
        pltpu.sync_copy(msg_hbm.at[pl.ds(wid * EPW * HID, EPW * HID)], msg_v)

        zero16 = jnp.zeros((LANES,), jnp.float32)

        @pl.loop(0, N * HID // LANES)
        def _(i):
            acc_v[pl.ds(i * LANES, LANES)] = zero16

        lane = lax.iota(jnp.int32, LANES)
        pair_sel = lax.shift_right_logical(lane, 3)
        col = lax.bitwise_and(lane, 7)

        @pl.loop(0, EPAIR)
        def _(j):
            rows = plsc.load_gather(dst_v, [j * 2 + pair_sel])
            flat = lax.shift_left(rows, 3) + col
            vals = msg_v[pl.ds(j * LANES, LANES)]
            plsc.addupdate_scatter(acc_v, [flat], vals)

        pltpu.sync_copy(acc_v, agg_hbm.at[wid])

    return body(msg_flat, dst)


def _msg_body(ea_ref, xj_ref, w1r_ref, b1r_ref, out_ref):
    ea = ea_ref[...]
    xj = xj_ref[...]
    cols = []
    for h in range(HID):
        ewh = jnp.maximum(
            jnp.dot(ea, w1r_ref[h], preferred_element_type=jnp.float32)
            + b1r_ref[h][None, :],
            0.0,
        )
        cols.append(jnp.sum(xj * ewh, axis=1, keepdims=True))
    out_ref[...] = jnp.concatenate(cols, axis=1)


def _messages(ea, xj, w1r, b1r):
    return pl.pallas_call(
        _msg_body,
        grid=(E // EB,),
        in_specs=[
            pl.BlockSpec((EB, D_E), lambda i: (i, 0)),
            pl.BlockSpec((EB, IN), lambda i: (i, 0)),
            pl.BlockSpec((HID, D_E, IN), lambda i: (0, 0, 0)),
            pl.BlockSpec((HID, IN), lambda i: (0, 0)),
        ],
        out_specs=pl.BlockSpec((EB, HID), lambda i: (i, 0)),
        out_shape=jax.ShapeDtypeStruct((E, HID), jnp.float32),
    )(ea, xj, w1r, b1r)


def _reduce_body(agg_ref, out_ref):
    out_ref[...] = jnp.sum(agg_ref[...], axis=0)


def _reduce_partials(agg):
    rows = N * HID // 128
    return pl.pallas_call(
        _reduce_body,
        grid=(1,),
        in_specs=[pl.BlockSpec((NW, rows, 128), lambda i: (0, 0, 0))],
        out_specs=pl.BlockSpec((rows, 128), lambda i: (0, 0)),
        out_shape=jax.ShapeDtypeStruct((rows, 128), jnp.float32),
    )(agg.reshape(NW, rows, 128))


def _node_body(x_ref, agg_ref, wr_ref, bc_ref, lg_ref, lb_ref, wl_ref,
               bl_ref, out_ref):
    x = x_ref[...]
    agg = agg_ref[...]
    h = (jnp.dot(x, wr_ref[...], preferred_element_type=jnp.float32)
         + agg + bc_ref[0][None, :])
    mu = jnp.mean(h, axis=1, keepdims=True)
    var = jnp.mean((h - mu) * (h - mu), axis=1, keepdims=True)
    hn = (h - mu) * lax.rsqrt(var + 1e-5) * lg_ref[0][None, :] + lb_ref[0][None, :]
    hn = jnp.maximum(hn, 0.0)
    out_ref[...] = (jnp.dot(hn, wl_ref[...], preferred_element_type=jnp.float32)
                    + bl_ref[0][None, :])


def _node_stage(x, agg, W_root, b_conv, ln_g, ln_b, W_lin, b_lin):
    return pl.pallas_call(
        _node_body,
        grid=(N // NB,),
        in_specs=[
            pl.BlockSpec((NB, IN), lambda i: (i, 0)),
            pl.BlockSpec((NB, HID), lambda i: (i, 0)),
            pl.BlockSpec((IN, HID), lambda i: (0, 0)),
            pl.BlockSpec((1, HID), lambda i: (0, 0)),
            pl.BlockSpec((1, HID), lambda i: (0, 0)),
            pl.BlockSpec((1, HID), lambda i: (0, 0)),
            pl.BlockSpec((HID, OUT), lambda i: (0, 0)),
            pl.BlockSpec((1, OUT), lambda i: (0, 0)),
        ],
        out_specs=pl.BlockSpec((NB, OUT), lambda i: (i, 0)),
        out_shape=jax.ShapeDtypeStruct((N, OUT), jnp.float32),
    )(x, agg, W_root, b_conv.reshape(1, HID), ln_g.reshape(1, HID),
      ln_b.reshape(1, HID), W_lin, b_lin.reshape(1, OUT))


def kernel(x, edge_attr, edge_index, W1, b1, W_root, b_conv, ln_g, ln_b,
           W_lin, b_lin):
    src = edge_index[0]
    dst = edge_index[1]
    w1r = W1.reshape(D_E, IN, HID).transpose(2, 0, 1)
    b1r = b1.reshape(IN, HID).T

    x_i32 = jax.lax.bitcast_convert_type(x, jnp.int32)
    xj_i32 = _gather_rows(x_i32, src)
    xj = jax.lax.bitcast_convert_type(xj_i32, jnp.float32)
    msg = _messages(edge_attr, xj, w1r, b1r)
    parts = _scatter_add(msg.reshape(E * HID), dst)
    agg = _reduce_partials(parts).reshape(N, HID)
    return _node_stage(x, agg, W_root, b_conv, ln_g, ln_b, W_lin, b_lin)


# speedup vs baseline: 1.1267x; 1.1267x over previous
import dataclasses
import functools

import jax
import jax.numpy as jnp
from jax import lax
from jax.experimental import pallas as pl
from jax.experimental.pallas import tpu as pltpu
from jax.experimental.pallas import tpu_sc as plsc

N = 10000
E = 160000
IN = 128
HID = 8
OUT = 128
D_E = 16

NC = 2
NS = 16
NW = NC * NS
EPW = E // NW
CH = 40
NCHUNK = EPW // CH

NPAD = 10240
ROWS_PER_TILE = NPAD // NS
MSG_W = 16

EB = 1280
NB = 2000

CHG = 128
NFULL = EPW // CHG
GTAIL = EPW - NFULL * CHG
ROW_W = IN


def _gather_rows(x_i32, src):
    mesh = plsc.VectorSubcoreMesh(core_axis_name="c", subcore_axis_name="s")

    @functools.partial(
        pl.kernel,
        mesh=mesh,
        out_type=jax.ShapeDtypeStruct((E, ROW_W), jnp.int32),
        scratch_types=[
            pltpu.VMEM((EPW,), jnp.int32),
            pltpu.VMEM((CHG, ROW_W), jnp.int32),
            pltpu.VMEM((GTAIL, ROW_W), jnp.int32),
        ],
    )
    def body(x_hbm, src_hbm, xj_hbm, idx_v, buf_v, tail_v):
        cid = lax.axis_index("c")
        sid = lax.axis_index("s")
        wid = sid * NC + cid
        base = wid * EPW

        pltpu.sync_copy(src_hbm.at[pl.ds(base, EPW)], idx_v)

        @pl.loop(0, NFULL)
        def _(k):
            pltpu.sync_copy(x_hbm.at[idx_v.at[pl.ds(k * CHG, CHG)]], buf_v)
            pltpu.sync_copy(buf_v, xj_hbm.at[pl.ds(base + k * CHG, CHG)])

        toff = NFULL * CHG
        pltpu.sync_copy(x_hbm.at[idx_v.at[pl.ds(toff, GTAIL)]], tail_v)
        pltpu.sync_copy(tail_v, xj_hbm.at[pl.ds(base + toff, GTAIL)])

    return body(x_i32, src)


def _scatter_add(msg_flat, dst):
    mesh = plsc.VectorSubcoreMesh(core_axis_name="c", subcore_axis_name="s")
    LANES = 16
    EPAIR = EPW // 2
    cp = pltpu.CompilerParams()
    if "needs_layout_passes" in pltpu.CompilerParams.__dataclass_fields__:
        cp = dataclasses.replace(cp, needs_layout_passes=False)

    @functools.partial(
        pl.kernel,
        mesh=mesh,
        compiler_params=cp,
        out_type=jax.ShapeDtypeStruct((NW, N * HID), jnp.float32),
        scratch_types=[
            pltpu.VMEM((EPW,), jnp.int32),
            pltpu.VMEM((EPW * HID,), jnp.float32),
            pltpu.VMEM((N * HID,), jnp.float32),
        ],
    )
    def body(msg_hbm, dst_hbm, agg_hbm, dst_v, msg_v, acc_v):
        cid = lax.axis_index("c")
        sid = lax.axis_index("s")
        wid = sid * NC + cid

        pltpu.sync_copy(dst_hbm.at[pl.ds(wid * EPW, EPW)], dst_v)
        pltpu.sync_copy(msg_hbm.at[pl.ds(wid * EPW * HID, EPW * HID)], msg_v)

        zero16 = jnp.zeros((LANES,), jnp.float32)

        @pl.loop(0, N * HID // LANES)
        def _(i):
            acc_v[pl.ds(i * LANES, LANES)] = zero16

        lane = lax.iota(jnp.int32, LANES)
        pair_sel = lax.shift_right_logical(lane, 3)
        col = lax.bitwise_and(lane, 7)

        @pl.loop(0, EPAIR)
        def _(j):
            rows = plsc.load_gather(dst_v, [j * 2 + pair_sel])
            flat = lax.shift_left(rows, 3) + col
            vals = msg_v[pl.ds(j * LANES, LANES)]
            plsc.addupdate_scatter(acc_v, [flat], vals)

        pltpu.sync_copy(acc_v, agg_hbm.at[wid])

    return body(msg_flat, dst)


def _msg_body(ea_ref, xj_ref, w1r_ref, b1r_ref, out_ref):
    ea = ea_ref[...]
    xj = xj_ref[...]
    cols = []
    for h in range(HID):
        ewh = jnp.maximum(
            jnp.dot(ea, w1r_ref[h], preferred_element_type=jnp.float32)
            + b1r_ref[h][None, :],
            0.0,
        )
        cols.append(jnp.sum(xj * ewh, axis=1, keepdims=True))
    out_ref[...] = jnp.concatenate(cols, axis=1)


def _messages(ea, xj, w1r, b1r):
    return pl.pallas_call(
        _msg_body,
        grid=(E // EB,),
        in_specs=[
            pl.BlockSpec((EB, D_E), lambda i: (i, 0)),
            pl.BlockSpec((EB, IN), lambda i: (i, 0)),
            pl.BlockSpec((HID, D_E, IN), lambda i: (0, 0, 0)),
            pl.BlockSpec((HID, IN), lambda i: (0, 0)),
        ],
        out_specs=pl.BlockSpec((EB, HID), lambda i: (i, 0)),
        out_shape=jax.ShapeDtypeStruct((E, HID), jnp.float32),
    )(ea, xj, w1r, b1r)


def _reduce_body(agg_ref, out_ref):
    out_ref[...] = jnp.sum(agg_ref[...], axis=0)


def _reduce_partials(agg):
    rows = N * HID // 128
    return pl.pallas_call(
        _reduce_body,
        grid=(1,),
        in_specs=[pl.BlockSpec((NW, rows, 128), lambda i: (0, 0, 0))],
        out_specs=pl.BlockSpec((rows, 128), lambda i: (0, 0)),
        out_shape=jax.ShapeDtypeStruct((rows, 128), jnp.float32),
    )(agg.reshape(NW, rows, 128))


def _node_body(x_ref, agg_ref, wr_ref, bc_ref, lg_ref, lb_ref, wl_ref,
               bl_ref, out_ref):
    x = x_ref[...]
    agg = agg_ref[...]
    h = (jnp.dot(x, wr_ref[...], preferred_element_type=jnp.float32)
         + agg + bc_ref[0][None, :])
    mu = jnp.mean(h, axis=1, keepdims=True)
    var = jnp.mean((h - mu) * (h - mu), axis=1, keepdims=True)
    hn = (h - mu) * lax.rsqrt(var + 1e-5) * lg_ref[0][None, :] + lb_ref[0][None, :]
    hn = jnp.maximum(hn, 0.0)
    out_ref[...] = (jnp.dot(hn, wl_ref[...], preferred_element_type=jnp.float32)
                    + bl_ref[0][None, :])


def _node_stage(x, agg, W_root, b_conv, ln_g, ln_b, W_lin, b_lin):
    return pl.pallas_call(
        _node_body,
        grid=(N // NB,),
        in_specs=[
            pl.BlockSpec((NB, IN), lambda i: (i, 0)),
            pl.BlockSpec((NB, HID), lambda i: (i, 0)),
            pl.BlockSpec((IN, HID), lambda i: (0, 0)),
            pl.BlockSpec((1, HID), lambda i: (0, 0)),
            pl.BlockSpec((1, HID), lambda i: (0, 0)),
            pl.BlockSpec((1, HID), lambda i: (0, 0)),
            pl.BlockSpec((HID, OUT), lambda i: (0, 0)),
            pl.BlockSpec((1, OUT), lambda i: (0, 0)),
        ],
        out_specs=pl.BlockSpec((NB, OUT), lambda i: (i, 0)),
        out_shape=jax.ShapeDtypeStruct((N, OUT), jnp.float32),
    )(x, agg, W_root, b_conv.reshape(1, HID), ln_g.reshape(1, HID),
      ln_b.reshape(1, HID), W_lin, b_lin.reshape(1, OUT))


def kernel(x, edge_attr, edge_index, W1, b1, W_root, b_conv, ln_g, ln_b,
           W_lin, b_lin):
    src = edge_index[0]
    dst = edge_index[1]
    w1r = W1.reshape(D_E, IN, HID).transpose(2, 0, 1)
    b1r = b1.reshape(IN, HID).T

    x_i32 = jax.lax.bitcast_convert_type(x, jnp.int32)
    xj_i32 = _gather_rows(x_i32, src)
    xj = jax.lax.bitcast_convert_type(xj_i32, jnp.float32)
    msg = _messages(edge_attr, xj, w1r, b1r)
    parts = _scatter_add(msg.reshape(E * HID), dst)
    agg = _reduce_partials(parts).reshape(N, HID)
    return _node_stage(x, agg, W_root, b_conv, ln_g, ln_b, W_lin, b_lin)


# speedup vs baseline: 1.1287x; 1.0017x over previous
import dataclasses
import functools

import jax
import jax.numpy as jnp
from jax import lax
from jax.experimental import pallas as pl
from jax.experimental.pallas import tpu as pltpu
from jax.experimental.pallas import tpu_sc as plsc

N = 10000
E = 160000
IN = 128
HID = 8
OUT = 128
D_E = 16

NC = 2
NS = 16
NW = NC * NS
EPW = E // NW

EB = 1280
NB = 2000

CHG = 128
NFULL = EPW // CHG
GTAIL = EPW - NFULL * CHG
ROW_W = IN


def _gather_rows(x_i32, src):
    mesh = plsc.VectorSubcoreMesh(core_axis_name="c", subcore_axis_name="s")

    @functools.partial(
        pl.kernel,
        mesh=mesh,
        out_type=jax.ShapeDtypeStruct((E, ROW_W), jnp.int32),
        scratch_types=[
            pltpu.VMEM((EPW,), jnp.int32),
            pltpu.VMEM((CHG, ROW_W), jnp.int32),
            pltpu.VMEM((GTAIL, ROW_W), jnp.int32),
        ],
    )
    def body(x_hbm, src_hbm, xj_hbm, idx_v, buf_v, tail_v):
        cid = lax.axis_index("c")
        sid = lax.axis_index("s")
        wid = sid * NC + cid
        base = wid * EPW

        pltpu.sync_copy(src_hbm.at[pl.ds(base, EPW)], idx_v)

        @pl.loop(0, NFULL)
        def _(k):
            pltpu.sync_copy(x_hbm.at[idx_v.at[pl.ds(k * CHG, CHG)]], buf_v)
            pltpu.sync_copy(buf_v, xj_hbm.at[pl.ds(base + k * CHG, CHG)])

        toff = NFULL * CHG
        pltpu.sync_copy(x_hbm.at[idx_v.at[pl.ds(toff, GTAIL)]], tail_v)
        pltpu.sync_copy(tail_v, xj_hbm.at[pl.ds(base + toff, GTAIL)])

    return body(x_i32, src)


def _scatter_add(msg_flat, dst):
    mesh = plsc.VectorSubcoreMesh(core_axis_name="c", subcore_axis_name="s")
    LANES = 16
    EPAIR = EPW // 2
    cp = pltpu.CompilerParams()
    if "needs_layout_passes" in pltpu.CompilerParams.__dataclass_fields__:
        cp = dataclasses.replace(cp, needs_layout_passes=False)

    @functools.partial(
        pl.kernel,
        mesh=mesh,
        compiler_params=cp,
        out_type=jax.ShapeDtypeStruct((NW, N * HID), jnp.float32),
        scratch_types=[
            pltpu.VMEM((EPW,), jnp.int32),
            pltpu.VMEM((EPW * HID,), jnp.float32),
            pltpu.VMEM((N * HID,), jnp.float32),
        ],
    )
    def body(msg_hbm, dst_hbm, agg_hbm, dst_v, msg_v, acc_v):
        cid = lax.axis_index("c")
        sid = lax.axis_index("s")
        wid = sid * NC + cid

        pltpu.sync_copy(dst_hbm.at[pl.ds(wid * EPW, EPW)], dst_v)
        pltpu.sync_copy(msg_hbm.at[pl.ds(wid * EPW * HID, EPW * HID)], msg_v)

        zero16 = jnp.zeros((LANES,), jnp.float32)

        @pl.loop(0, N * HID // LANES)
        def _(i):
            acc_v[pl.ds(i * LANES, LANES)] = zero16

        lane = lax.iota(jnp.int32, LANES)
        pair_sel = lax.shift_right_logical(lane, 3)
        col = lax.bitwise_and(lane, 7)

        @pl.loop(0, EPAIR)
        def _(j):
            rows = plsc.load_gather(dst_v, [j * 2 + pair_sel])
            flat = lax.shift_left(rows, 3) + col
            vals = msg_v[pl.ds(j * LANES, LANES)]
            plsc.addupdate_scatter(acc_v, [flat], vals)

        pltpu.sync_copy(acc_v, agg_hbm.at[wid])

    return body(msg_flat, dst)


def _msg_body(ea_ref, xj_ref, w1r_ref, b1r_ref, out_ref):
    ea = ea_ref[...]
    xj = xj_ref[...]
    cols = []
    for h in range(HID):
        ewh = jnp.maximum(
            jnp.dot(ea, w1r_ref[h], preferred_element_type=jnp.float32)
            + b1r_ref[h][None, :],
            0.0,
        )
        cols.append(jnp.sum(xj * ewh, axis=1, keepdims=True))
    out_ref[...] = jnp.concatenate(cols, axis=1)


def _messages(ea, xj, w1r, b1r):
    return pl.pallas_call(
        _msg_body,
        grid=(E // EB,),
        in_specs=[
            pl.BlockSpec((EB, D_E), lambda i: (i, 0)),
            pl.BlockSpec((EB, IN), lambda i: (i, 0)),
            pl.BlockSpec((HID, D_E, IN), lambda i: (0, 0, 0)),
            pl.BlockSpec((HID, IN), lambda i: (0, 0)),
        ],
        out_specs=pl.BlockSpec((EB, HID), lambda i: (i, 0)),
        out_shape=jax.ShapeDtypeStruct((E, HID), jnp.float32),
    )(ea, xj, w1r, b1r)


def _reduce_body(agg_ref, out_ref):
    out_ref[...] = jnp.sum(agg_ref[...], axis=0)


def _reduce_partials(agg):
    rows = N * HID // 128
    return pl.pallas_call(
        _reduce_body,
        grid=(1,),
        in_specs=[pl.BlockSpec((NW, rows, 128), lambda i: (0, 0, 0))],
        out_specs=pl.BlockSpec((rows, 128), lambda i: (0, 0)),
        out_shape=jax.ShapeDtypeStruct((rows, 128), jnp.float32),
    )(agg.reshape(NW, rows, 128))


def _node_body(x_ref, agg_ref, wr_ref, bc_ref, lg_ref, lb_ref, wl_ref,
               bl_ref, out_ref):
    x = x_ref[...]
    agg = agg_ref[...]
    h = (jnp.dot(x, wr_ref[...], preferred_element_type=jnp.float32)
         + agg + bc_ref[0][None, :])
    mu = jnp.mean(h, axis=1, keepdims=True)
    var = jnp.mean((h - mu) * (h - mu), axis=1, keepdims=True)
    hn = (h - mu) * lax.rsqrt(var + 1e-5) * lg_ref[0][None, :] + lb_ref[0][None, :]
    hn = jnp.maximum(hn, 0.0)
    out_ref[...] = (jnp.dot(hn, wl_ref[...], preferred_element_type=jnp.float32)
                    + bl_ref[0][None, :])


def _node_stage(x, agg, W_root, b_conv, ln_g, ln_b, W_lin, b_lin):
    return pl.pallas_call(
        _node_body,
        grid=(N // NB,),
        in_specs=[
            pl.BlockSpec((NB, IN), lambda i: (i, 0)),
            pl.BlockSpec((NB, HID), lambda i: (i, 0)),
            pl.BlockSpec((IN, HID), lambda i: (0, 0)),
            pl.BlockSpec((1, HID), lambda i: (0, 0)),
            pl.BlockSpec((1, HID), lambda i: (0, 0)),
            pl.BlockSpec((1, HID), lambda i: (0, 0)),
            pl.BlockSpec((HID, OUT), lambda i: (0, 0)),
            pl.BlockSpec((1, OUT), lambda i: (0, 0)),
        ],
        out_specs=pl.BlockSpec((NB, OUT), lambda i: (i, 0)),
        out_shape=jax.ShapeDtypeStruct((N, OUT), jnp.float32),
    )(x, agg, W_root, b_conv.reshape(1, HID), ln_g.reshape(1, HID),
      ln_b.reshape(1, HID), W_lin, b_lin.reshape(1, OUT))


def kernel(x, edge_attr, edge_index, W1, b1, W_root, b_conv, ln_g, ln_b,
           W_lin, b_lin):
    src = edge_index[0]
    dst = edge_index[1]
    w1r = W1.reshape(D_E, IN, HID).transpose(2, 0, 1)
    b1r = b1.reshape(IN, HID).T

    x_i32 = jax.lax.bitcast_convert_type(x, jnp.int32)
    xj_i32 = _gather_rows(x_i32, src)
    xj = jax.lax.bitcast_convert_type(xj_i32, jnp.float32)
    msg = _messages(edge_attr, xj, w1r, b1r)
    parts = _scatter_add(msg.reshape(E * HID), dst)
    agg = _reduce_partials(parts).reshape(N, HID)
    return _node_stage(x, agg, W_root, b_conv, ln_g, ln_b, W_lin, b_lin)
